# baseline (speedup 1.0000x reference)
import functools

import jax
import jax.numpy as jnp
from jax import lax
from jax.experimental import pallas as pl
from jax.experimental.pallas import tpu as pltpu
from jax.experimental.pallas import tpu_sc as plsc

LEN_USERS = 100000
EMBED_DIM = 128
BATCH = 16384

NUM_CORES = 2
NUM_SUBCORES = 16
LANES = 16
NUM_WORKERS = NUM_CORES * NUM_SUBCORES
BPW = BATCH // NUM_WORKERS
IDX_CHUNK = 128
NCHUNK = BPW // IDX_CHUNK
CVEC = EMBED_DIM // LANES


def _sc_kernel_body(x_hbm, table_hbm, w_hbm, b_hbm, out_hbm,
                    x_v, idx_v, rows_v, mov_v, acc_v, w_v, b_v,
                    sem_x, sem_w, sem_b, sem_g0, sem_g1, sem_g2, sem_g3):
    wid = lax.axis_index("s") * NUM_CORES + lax.axis_index("c")
    base = wid * BPW

    cp_x = pltpu.async_copy(x_hbm.at[pl.ds(2 * base, 2 * BPW)], x_v, sem_x)
    cp_w = pltpu.async_copy(w_hbm, w_v.at[pl.ds(0, 1)], sem_w)
    cp_b = pltpu.async_copy(b_hbm, b_v.at[pl.ds(0, 1)], sem_b)
    cp_x.wait()

    lane = lax.iota(jnp.int32, LANES)
    lane2 = lane * 2

    for t in range(BPW // LANES):
        u = plsc.load_gather(x_v, [lane2 + (2 * LANES) * t])
        idx_v[t // CVEC, pl.ds((t % CVEC) * LANES, LANES)] = u

    gsems = [sem_g0, sem_g1, sem_g2, sem_g3]
    copies = [
        pltpu.async_copy(table_hbm.at[idx_v.at[j]],
                         rows_v.at[pl.ds(j * IDX_CHUNK, IDX_CHUNK)], gsems[j])
        for j in range(NCHUNK)
    ]

    for t in range(BPW // LANES):
        m = plsc.load_gather(x_v, [lane2 + (2 * LANES) * t + 1])
        mov_v[pl.ds(t * LANES, LANES)] = m.astype(jnp.float32)

    cp_w.wait()
    cp_b.wait()
    zero_idx = jnp.zeros((LANES,), jnp.int32)
    w_vec = w_v[...].at[zero_idx].get(mode="promise_in_bounds")
    b_vec = b_v[...].at[zero_idx].get(mode="promise_in_bounds")

    lane_masks = [lane == j for j in range(LANES)]
    last = jnp.full((LANES,), LANES - 1, jnp.int32)
    one = jnp.ones((LANES,), jnp.float32)

    def make_group_body(chunk):
        def group_body(g, _):
            row0 = chunk * IDX_CHUNK + g * LANES
            res = jnp.zeros((LANES,), jnp.float32)
            for j in range(LANES):
                acc = rows_v[row0 + j, pl.ds(0, LANES)]
                for c in range(1, CVEC):
                    acc = acc + rows_v[row0 + j, pl.ds(c * LANES, LANES)]
                s_vec = plsc.cumsum(acc).at[last].get(mode="promise_in_bounds")
                res = jnp.where(lane_masks[j], s_vec, res)
            z = res * mov_v[pl.ds(row0, LANES)] * w_vec + b_vec
            t = jnp.exp(-jnp.abs(z))
            acc_v[pl.ds(row0, LANES)] = jnp.where(
                z >= 0, one / (one + t), t / (one + t))
            return 0
        return group_body

    for j in range(NCHUNK):
        copies[j].wait()
        lax.fori_loop(0, IDX_CHUNK // LANES, make_group_body(j), 0)

    pltpu.sync_copy(acc_v, out_hbm.at[pl.ds(base, BPW)])


@jax.jit
def kernel(x, u_table, W, b):
    mesh = plsc.VectorSubcoreMesh(core_axis_name="c", subcore_axis_name="s",
                                  num_cores=NUM_CORES,
                                  num_subcores=NUM_SUBCORES)
    run = functools.partial(
        pl.kernel,
        out_type=jax.ShapeDtypeStruct((BATCH,), jnp.float32),
        mesh=mesh,
        compiler_params=pltpu.CompilerParams(needs_layout_passes=False),
        scratch_types=[
            pltpu.VMEM((2 * BPW,), jnp.int32),
            pltpu.VMEM((NCHUNK, IDX_CHUNK), jnp.int32),
            pltpu.VMEM((BPW, EMBED_DIM), jnp.float32),
            pltpu.VMEM((BPW,), jnp.float32),
            pltpu.VMEM((BPW,), jnp.float32),
            pltpu.VMEM((LANES,), jnp.float32),
            pltpu.VMEM((LANES,), jnp.float32),
        ] + [pltpu.SemaphoreType.DMA] * 7,
    )(_sc_kernel_body)
    out = run(x.reshape(-1), u_table, W.reshape(-1), b)
    return out.reshape(BATCH, 1)


# speedup vs baseline: 1.2610x; 1.2610x over previous
import functools

import jax
import jax.numpy as jnp
from jax import lax
from jax.experimental import pallas as pl
from jax.experimental.pallas import tpu as pltpu
from jax.experimental.pallas import tpu_sc as plsc

LEN_USERS = 100000
EMBED_DIM = 128
BATCH = 16384

NUM_CORES = 2
NUM_SUBCORES = 16
LANES = 16
NUM_WORKERS = NUM_CORES * NUM_SUBCORES
BPW = BATCH // NUM_WORKERS
IDX_CHUNK = 128
NCHUNK = BPW // IDX_CHUNK
CVEC = EMBED_DIM // LANES


def _sc_kernel_body(uid_hbm, mov_hbm, table_hbm, wb_hbm, out_hbm,
                    idx_v, rows_v, mov_v, acc_v, wb_v,
                    sem_i, sem_m, sem_w, sem_g0, sem_g1, sem_g2, sem_g3):
    wid = lax.axis_index("s") * NUM_CORES + lax.axis_index("c")
    base = wid * BPW

    cp_i = pltpu.async_copy(uid_hbm.at[pl.ds(base, BPW)], idx_v, sem_i)
    cp_m = pltpu.async_copy(mov_hbm.at[pl.ds(base, BPW)], mov_v, sem_m)
    cp_w = pltpu.async_copy(wb_hbm, wb_v, sem_w)
    cp_i.wait()

    gsems = [sem_g0, sem_g1, sem_g2, sem_g3]
    copies = [
        pltpu.async_copy(table_hbm.at[idx_v.at[pl.ds(j * IDX_CHUNK, IDX_CHUNK)]],
                         rows_v.at[pl.ds(j * IDX_CHUNK, IDX_CHUNK)], gsems[j])
        for j in range(NCHUNK)
    ]

    cp_m.wait()
    cp_w.wait()
    w_vec = wb_v[pl.ds(0, LANES)]
    b_vec = wb_v[pl.ds(LANES, LANES)]

    lane = lax.iota(jnp.int32, LANES)
    lane_masks = [lane == j for j in range(LANES)]
    last = jnp.full((LANES,), LANES - 1, jnp.int32)
    one = jnp.ones((LANES,), jnp.float32)

    def make_group_body(chunk):
        def group_body(g, _):
            row0 = chunk * IDX_CHUNK + g * LANES
            res = jnp.zeros((LANES,), jnp.float32)
            for j in range(LANES):
                acc = rows_v[row0 + j, pl.ds(0, LANES)]
                for c in range(1, CVEC):
                    acc = acc + rows_v[row0 + j, pl.ds(c * LANES, LANES)]
                s_vec = plsc.cumsum(acc).at[last].get(mode="promise_in_bounds")
                res = jnp.where(lane_masks[j], s_vec, res)
            z = res * mov_v[pl.ds(row0, LANES)] * w_vec + b_vec
            t = jnp.exp(-jnp.abs(z))
            acc_v[pl.ds(row0, LANES)] = jnp.where(
                z >= 0, one / (one + t), t / (one + t))
            return 0
        return group_body

    for j in range(NCHUNK):
        copies[j].wait()
        lax.fori_loop(0, IDX_CHUNK // LANES, make_group_body(j), 0)

    pltpu.sync_copy(acc_v, out_hbm.at[pl.ds(base, BPW)])


@jax.jit
def kernel(x, u_table, W, b):
    uid = x[:, 0]
    mov = x[:, 1].astype(jnp.float32)
    wb = jnp.concatenate([jnp.full((LANES,), W[0, 0], jnp.float32),
                          jnp.full((LANES,), b[0], jnp.float32)])

    mesh = plsc.VectorSubcoreMesh(core_axis_name="c", subcore_axis_name="s",
                                  num_cores=NUM_CORES,
                                  num_subcores=NUM_SUBCORES)
    run = functools.partial(
        pl.kernel,
        out_type=jax.ShapeDtypeStruct((BATCH,), jnp.float32),
        mesh=mesh,
        compiler_params=pltpu.CompilerParams(needs_layout_passes=False),
        scratch_types=[
            pltpu.VMEM((BPW,), jnp.int32),
            pltpu.VMEM((BPW, EMBED_DIM), jnp.float32),
            pltpu.VMEM((BPW,), jnp.float32),
            pltpu.VMEM((BPW,), jnp.float32),
            pltpu.VMEM((2 * LANES,), jnp.float32),
        ] + [pltpu.SemaphoreType.DMA] * 7,
    )(_sc_kernel_body)
    out = run(uid, mov, u_table, wb)
    return out.reshape(BATCH, 1)
